# baseline (speedup 1.0000x reference)
import functools

import jax
import jax.numpy as jnp
from jax import lax
from jax.experimental import pallas as pl
from jax.experimental.pallas import tpu as pltpu
from jax.experimental.pallas import tpu_sc as plsc

NC = 2
NS = 16
NW = NC * NS
L = 16
M = 65536
D = 128
RPW = M // NW
IMAX = 2147483647

BLK = 2048
NBLK = M // BLK

_mesh = plsc.VectorSubcoreMesh(core_axis_name="c", subcore_axis_name="s")
_params = pltpu.CompilerParams(needs_layout_passes=False)


def _scan_body(q_ref, k_ref, a_ref, bm_ref, bi_ref):
    b = pl.program_id(0)
    qv = q_ref[0, :]
    s = jnp.dot(k_ref[...], qv, preferred_element_type=jnp.float32)
    s = jnp.where(a_ref[...] > 0.0, s, -jnp.inf)
    m = jnp.max(s)
    io = lax.broadcasted_iota(jnp.int32, (BLK,), 0) + b * BLK
    idx = jnp.min(jnp.where(s == m, io, IMAX))
    bm_ref[...] = jnp.broadcast_to(m, (1, 1, D))
    bi_ref[...] = jnp.broadcast_to(idx, (1, 1, D))


_scan = pl.pallas_call(
    _scan_body,
    grid=(NBLK,),
    in_specs=[
        pl.BlockSpec((1, D), lambda b: (0, 0)),
        pl.BlockSpec((BLK, D), lambda b: (b, 0)),
        pl.BlockSpec((BLK,), lambda b: (b,)),
    ],
    out_specs=[
        pl.BlockSpec((1, 1, D), lambda b: (b, 0, 0)),
        pl.BlockSpec((1, 1, D), lambda b: (b, 0, 0)),
    ],
    out_shape=[
        jax.ShapeDtypeStruct((NBLK, 1, D), jnp.float32),
        jax.ShapeDtypeStruct((NBLK, 1, D), jnp.int32),
    ],
)


def _finalize_body(cm_hbm, ci_hbm, uc_hbm, pc_hbm, av_hbm,
                   ucout_hbm, act_hbm, outf_hbm, outi_hbm,
                   cmv, civ, ucb, pcv, idxv, abuf, sv, si, sem, semu):
    cid = lax.axis_index("c")
    sid = lax.axis_index("s")
    wid = sid * NC + cid
    wbase = wid * RPW

    pltpu.async_copy(uc_hbm.at[pl.ds(wbase, RPW)], ucb, semu)
    pltpu.sync_copy(cm_hbm, cmv)
    pltpu.sync_copy(ci_hbm, civ)
    pltpu.sync_copy(pc_hbm, pcv)

    iota = lax.iota(jnp.int32, L)
    stride = iota * D
    v0 = plsc.load_gather(cmv, [stride])
    v1 = plsc.load_gather(cmv, [stride + L * D])
    i0 = plsc.load_gather(civ, [stride])
    i1 = plsc.load_gather(civ, [stride + L * D])
    gm = jnp.max(jnp.maximum(v0, v1))
    c0 = jnp.where(v0 == gm, i0, IMAX)
    c1 = jnp.where(v1 == gm, i1, IMAX)
    gi = jnp.minimum(jnp.min(c0), jnp.min(c1))
    gi = jnp.where(gi == IMAX, 0, gi)
    hit = gm > 0.95

    pltpu.make_async_copy(uc_hbm.at[pl.ds(wbase, RPW)], ucb, semu).wait()
    off = jnp.clip(gi - wbase, 0, RPW - 1)
    own = (iota == 0) & jnp.broadcast_to(
        hit & (gi >= wbase) & (gi < wbase + RPW), (L,))
    plsc.addupdate_scatter(
        ucb, [jnp.broadcast_to(off, (L,))], jnp.ones((L,), jnp.int32),
        mask=own)
    pltpu.sync_copy(ucb, ucout_hbm.at[pl.ds(wbase, RPW)])

    @pl.when(wid == 0)
    def _():
        giv = jnp.broadcast_to(gi, (L,))
        idxv[...] = giv
        pltpu.async_copy(av_hbm.at[idxv], abuf, sem).wait()
        pltpu.sync_copy(abuf.at[0], act_hbm)
        sv[...] = jnp.broadcast_to(gm, (L,))
        pltpu.sync_copy(sv, outf_hbm)
        newpc = jnp.where(jnp.broadcast_to(hit, (L,)), giv, pcv[...])
        si[...] = jnp.where(iota == 0, giv, jnp.where(iota == 1, newpc, 0))
        pltpu.sync_copy(si, outi_hbm)


_finalize = functools.partial(
    pl.kernel,
    out_type=(
        jax.ShapeDtypeStruct((M,), jnp.int32),
        jax.ShapeDtypeStruct((D,), jnp.float32),
        jax.ShapeDtypeStruct((L,), jnp.float32),
        jax.ShapeDtypeStruct((L,), jnp.int32),
    ),
    mesh=_mesh,
    compiler_params=_params,
    scratch_types=[
        pltpu.VMEM((NBLK * D,), jnp.float32),
        pltpu.VMEM((NBLK * D,), jnp.int32),
        pltpu.VMEM((RPW,), jnp.int32),
        pltpu.VMEM((L,), jnp.int32),
        pltpu.VMEM((L,), jnp.int32),
        pltpu.VMEM((L, D), jnp.float32),
        pltpu.VMEM((L,), jnp.float32),
        pltpu.VMEM((L,), jnp.int32),
        pltpu.SemaphoreType.DMA,
        pltpu.SemaphoreType.DMA,
    ],
)(_finalize_body)


def kernel(sensor_spikes, sensor_keys, action_values, is_active,
           usage_counts, program_counter):
    maskf = is_active.astype(jnp.float32)
    pc16 = jnp.broadcast_to(program_counter, (L,)).astype(jnp.int32)

    cm, ci = _scan(sensor_spikes, sensor_keys, maskf)
    ucn, act, outf, outi = _finalize(
        cm.reshape(NBLK * D), ci.reshape(NBLK * D),
        usage_counts, pc16, action_values)

    action = act.reshape(1, D)
    confidence = outf[0:1]
    best_idx = outi[0:1]
    new_pc = outi[1]
    return action, confidence, best_idx, ucn, new_pc


# speedup vs baseline: 1.1989x; 1.1989x over previous
import functools

import jax
import jax.numpy as jnp
from jax import lax
from jax.experimental import pallas as pl
from jax.experimental.pallas import tpu as pltpu
from jax.experimental.pallas import tpu_sc as plsc

NC = 2
NS = 16
NW = NC * NS
L = 16
M = 65536
D = 128
RPW = M // NW
IMAX = 2147483647

BLK = 2048
NBLK = M // BLK

_mesh = plsc.VectorSubcoreMesh(core_axis_name="c", subcore_axis_name="s")
_params = pltpu.CompilerParams(needs_layout_passes=False)


def _scan_body(q_ref, k_ref, a_ref, bm_ref, bi_ref):
    b = pl.program_id(0)
    s = jax.lax.dot_general(
        q_ref[...], k_ref[...],
        dimension_numbers=(((1,), (1,)), ((), ())),
        preferred_element_type=jnp.float32)
    s = jnp.where(a_ref[...] > 0.0, s, -jnp.inf)
    m = jnp.max(s)
    io = lax.broadcasted_iota(jnp.int32, (1, BLK), 1) + b * BLK
    idx = jnp.min(jnp.where(s == m, io, IMAX))
    bm_ref[...] = jnp.broadcast_to(m, (1, 1, D))
    bi_ref[...] = jnp.broadcast_to(idx, (1, 1, D))


_scan = pl.pallas_call(
    _scan_body,
    grid=(NBLK,),
    in_specs=[
        pl.BlockSpec((1, D), lambda b: (0, 0)),
        pl.BlockSpec((BLK, D), lambda b: (b, 0)),
        pl.BlockSpec((1, BLK), lambda b: (0, b)),
    ],
    out_specs=[
        pl.BlockSpec((1, 1, D), lambda b: (b, 0, 0)),
        pl.BlockSpec((1, 1, D), lambda b: (b, 0, 0)),
    ],
    out_shape=[
        jax.ShapeDtypeStruct((NBLK, 1, D), jnp.float32),
        jax.ShapeDtypeStruct((NBLK, 1, D), jnp.int32),
    ],
)


def _finalize_body(cm_hbm, ci_hbm, uc_hbm, pc_hbm, av_hbm,
                   ucout_hbm, act_hbm, outf_hbm, outi_hbm,
                   cmv, civ, ucb, pcv, idxv, abuf, sv, si, sem, semu):
    cid = lax.axis_index("c")
    sid = lax.axis_index("s")
    wid = sid * NC + cid
    wbase = wid * RPW

    pltpu.async_copy(uc_hbm.at[pl.ds(wbase, RPW)], ucb, semu)
    pltpu.sync_copy(cm_hbm, cmv)
    pltpu.sync_copy(ci_hbm, civ)
    pltpu.sync_copy(pc_hbm, pcv)

    iota = lax.iota(jnp.int32, L)
    stride = iota * D
    v0 = plsc.load_gather(cmv, [stride])
    v1 = plsc.load_gather(cmv, [stride + L * D])
    i0 = plsc.load_gather(civ, [stride])
    i1 = plsc.load_gather(civ, [stride + L * D])
    gm = jnp.max(jnp.maximum(v0, v1))
    c0 = jnp.where(v0 == gm, i0, IMAX)
    c1 = jnp.where(v1 == gm, i1, IMAX)
    gi = jnp.minimum(jnp.min(c0), jnp.min(c1))
    gi = jnp.where(gi == IMAX, 0, gi)
    hit = gm > 0.95

    pltpu.make_async_copy(uc_hbm.at[pl.ds(wbase, RPW)], ucb, semu).wait()
    off = jnp.clip(gi - wbase, 0, RPW - 1)
    own = (iota == 0) & jnp.broadcast_to(
        hit & (gi >= wbase) & (gi < wbase + RPW), (L,))
    plsc.addupdate_scatter(
        ucb, [jnp.broadcast_to(off, (L,))], jnp.ones((L,), jnp.int32),
        mask=own)
    pltpu.sync_copy(ucb, ucout_hbm.at[pl.ds(wbase, RPW)])

    @pl.when(wid == 0)
    def _():
        giv = jnp.broadcast_to(gi, (L,))
        idxv[...] = giv
        pltpu.async_copy(av_hbm.at[idxv], abuf, sem).wait()
        pltpu.sync_copy(abuf.at[0], act_hbm)
        sv[...] = jnp.broadcast_to(gm, (L,))
        pltpu.sync_copy(sv, outf_hbm)
        newpc = jnp.where(jnp.broadcast_to(hit, (L,)), giv, pcv[...])
        si[...] = jnp.where(iota == 0, giv, jnp.where(iota == 1, newpc, 0))
        pltpu.sync_copy(si, outi_hbm)


_finalize = functools.partial(
    pl.kernel,
    out_type=(
        jax.ShapeDtypeStruct((M,), jnp.int32),
        jax.ShapeDtypeStruct((D,), jnp.float32),
        jax.ShapeDtypeStruct((L,), jnp.float32),
        jax.ShapeDtypeStruct((L,), jnp.int32),
    ),
    mesh=_mesh,
    compiler_params=_params,
    scratch_types=[
        pltpu.VMEM((NBLK * D,), jnp.float32),
        pltpu.VMEM((NBLK * D,), jnp.int32),
        pltpu.VMEM((RPW,), jnp.int32),
        pltpu.VMEM((L,), jnp.int32),
        pltpu.VMEM((L,), jnp.int32),
        pltpu.VMEM((L, D), jnp.float32),
        pltpu.VMEM((L,), jnp.float32),
        pltpu.VMEM((L,), jnp.int32),
        pltpu.SemaphoreType.DMA,
        pltpu.SemaphoreType.DMA,
    ],
)(_finalize_body)


def kernel(sensor_spikes, sensor_keys, action_values, is_active,
           usage_counts, program_counter):
    maskf = is_active.astype(jnp.float32).reshape(1, M)
    pc16 = jnp.broadcast_to(program_counter, (L,)).astype(jnp.int32)

    cm, ci = _scan(sensor_spikes, sensor_keys, maskf)
    ucn, act, outf, outi = _finalize(
        cm.reshape(NBLK * D), ci.reshape(NBLK * D),
        usage_counts, pc16, action_values)

    action = act.reshape(1, D)
    confidence = outf[0:1]
    best_idx = outi[0:1]
    new_pc = outi[1]
    return action, confidence, best_idx, ucn, new_pc
